# baseline (speedup 1.0000x reference)
import jax
import jax.numpy as jnp
from jax import lax
from jax.experimental import pallas as pl
from jax.experimental.pallas import tpu as pltpu
from jax.experimental.pallas import tpu_sc as plsc

_NUM_MAPPERS = 1000000
_EMBED_DIM = 64
_PAD_DIM = 128
_BATCH = 16384
_TRASH = _BATCH

_NC = 2
_NS = 16
_NCOLS = 7813
_CPW = 245
_CCH = 4
_CHW = _CCH * 128
_NCH = 62
_MAXC = _NCOLS - _CCH
_CAP = _BATCH + 16

_LANES = None


def _rsqrt(x):
    i = lax.bitcast_convert_type(x, jnp.int32)
    i = jnp.int32(0x5F3759DF) - lax.shift_right_arithmetic(i, 1)
    y = lax.bitcast_convert_type(i, jnp.float32)
    half = jnp.float32(0.5) * x
    for _ in range(3):
        y = y * (jnp.float32(1.5) - half * y * y)
    return y


def _shift_gather(v, idx):
    dnums = lax.GatherDimensionNumbers(
        offset_dims=(), collapsed_slice_dims=(0,), start_index_map=(0,))
    return lax.gather(v, idx[:, None], dnums, (1,),
                      mode=lax.GatherScatterMode.PROMISE_IN_BOUNDS)


def _excl_prefix_and_total(m, lanes):
    v = jnp.where(m, jnp.int32(1), jnp.int32(0))
    incl = v
    for s in (1, 2, 4, 8):
        shifted = _shift_gather(incl, jnp.maximum(lanes - s, 0))
        incl = incl + jnp.where(lanes >= s, shifted, jnp.int32(0))
    total_splat = _shift_gather(incl, jnp.broadcast_to(jnp.int32(15), (16,)))
    return incl - v, total_splat


def _compact(list_ref, off, packed, m, lanes):
    excl, tot = _excl_prefix_and_total(m, lanes)
    plsc.store_scatter(list_ref, [off + excl], packed, mask=m)
    return off + tot[0]


def _embed_body(ids_hbm, tab3_hbm, gamma_hbm, beta_hbm, out2_hbm,
                ids_v, selid_v, hit_v,
                stage_a, stage_b, bank_v, posb_v, gamma_v, beta_v,
                sem, osem):
    wid = lax.axis_index("s") * _NC + lax.axis_index("c")
    lanes = lax.iota(jnp.int32, 16)

    pltpu.sync_copy(ids_hbm, ids_v)
    pltpu.sync_copy(gamma_hbm, gamma_v)
    pltpu.sync_copy(beta_hbm, beta_v)

    lo_col = wid * _CPW
    hi_col = jnp.minimum(lo_col + _CPW, jnp.int32(_NCOLS))
    lo = lo_col * jnp.int32(128)
    hi = hi_col * jnp.int32(128)

    def sel_body(i, off):
        v = ids_v[pl.ds(i * 16, 16)]
        v = jnp.where(v == jnp.int32(-1), jnp.int32(_NUM_MAPPERS), v)
        v = jnp.minimum(jnp.maximum(v, jnp.int32(0)),
                        jnp.int32(_NUM_MAPPERS))
        m = (v >= lo) & (v < hi)
        packed = lax.shift_left(i * 16 + lanes, jnp.int32(15)) | (v - lo)
        return _compact(selid_v, off, packed, m, lanes)

    total = lax.fori_loop(0, _BATCH // 16, sel_body, jnp.int32(0))
    nvec = (total + jnp.int32(15)) // jnp.int32(16)

    g_vec = [gamma_v[pl.ds(16 * q, 16)] for q in range(4)]
    b_vec = [beta_v[pl.ds(16 * q, 16)] for q in range(4)]
    inv_d = jnp.float32(1.0 / _EMBED_DIM)
    eps = jnp.float32(1e-5)

    for b in range(2):
        posb_v[b, pl.ds(0, 16)] = jnp.broadcast_to(jnp.int32(_TRASH), (16,))
        pltpu.async_copy(bank_v.at[b], out2_hbm.at[posb_v.at[b]], osem)

    def chunk_base(j):
        return (jnp.minimum(lo_col + _CCH * j, jnp.int32(_MAXC))
                * jnp.int32(128))

    def fire(j, buf):
        cb = pl.multiple_of(chunk_base(j), 128)
        return pltpu.async_copy(tab3_hbm.at[:, :, pl.ds(cb, _CHW)], buf,
                                sem)

    def drain(buf):
        pltpu.make_async_copy(tab3_hbm.at[:, :, pl.ds(0, _CHW)],
                              buf, sem).wait()

    def gath(buf, a, b, local):
        return plsc.load_gather(
            buf, [jnp.broadcast_to(jnp.int32(a), (16,)),
                  jnp.broadcast_to(jnp.int32(b), (16,)), local])

    def process(j, buf, g_fired):
        cb = chunk_base(j)
        cbr = cb - lo

        def scan_body(s, hoff):
            spk = selid_v[pl.ds(s * 16, 16)]
            rel = spk & jnp.int32(0x7FFF)
            pv = lax.shift_right_logical(spk, jnp.int32(15))
            m = (((s * 16 + lanes) < total)
                 & (rel >= cbr) & (rel < cbr + jnp.int32(_CHW)))
            packed = lax.shift_left(pv, jnp.int32(9)) | (rel - cbr)
            return _compact(hit_v, hoff, packed, m, lanes)

        nh = lax.fori_loop(0, nvec, scan_body, jnp.int32(0))
        hvec = (nh + jnp.int32(15)) // jnp.int32(16)

        def hit_body(h, g):
            slot = g % jnp.int32(2)
            pltpu.make_async_copy(bank_v.at[0],
                                  out2_hbm.at[posb_v.at[0]], osem).wait()
            hpk = hit_v[pl.ds(h * 16, 16)]
            hpv = lax.shift_right_logical(hpk, jnp.int32(9))
            live = (h * 16 + lanes) < nh
            local = jnp.minimum(
                jnp.maximum(hpk & jnp.int32(_CHW - 1), jnp.int32(0)),
                jnp.int32(_CHW - 1))
            slot16 = jnp.broadcast_to(slot, (16,))
            acc_s = jnp.zeros((16,), jnp.float32)
            acc_q = jnp.zeros((16,), jnp.float32)
            cols = []
            for f in range(_EMBED_DIM):
                gv = gath(buf, f // 8, f % 8, local)
                cols.append(gv)
                acc_s = acc_s + gv
                acc_q = acc_q + gv * gv
            mean = acc_s * inv_d
            var = acc_q * inv_d - mean * mean
            rv = _rsqrt(var + eps)
            for f in range(_EMBED_DIM):
                gf = g_vec[f // 16][f % 16]
                bf = b_vec[f // 16][f % 16]
                n = (cols[f] - mean) * rv * gf + bf
                plsc.store_scatter(
                    bank_v,
                    [slot16, lanes,
                     jnp.broadcast_to(jnp.int32(f), (16,))], n)
            plsc.store_scatter(
                posb_v, [slot16, lanes],
                jnp.where(live, hpv, jnp.int32(_TRASH)))
            pltpu.async_copy(bank_v.at[slot], out2_hbm.at[posb_v.at[slot]],
                             osem)
            return g + jnp.int32(1)

        return lax.fori_loop(0, hvec, hit_body, g_fired)

    fire(0, stage_a)
    fire(1, stage_b)

    def pair_body(jj, g):
        ja = jj * 2
        drain(stage_a)
        g = process(ja, stage_a, g)
        fire(ja + 2, stage_a)
        drain(stage_b)
        g = process(ja + 1, stage_b, g)
        fire(ja + 3, stage_b)
        return g

    lax.fori_loop(0, _NCH // 2, pair_body, jnp.int32(0))
    drain(stage_a)
    drain(stage_b)
    for b in range(2):
        pltpu.make_async_copy(bank_v.at[0], out2_hbm.at[posb_v.at[0]],
                              osem).wait()


@jax.jit
def _embed(mapper_ids, table, ln_gamma, ln_beta):
    mesh = plsc.VectorSubcoreMesh(core_axis_name="c", subcore_axis_name="s")
    f = pl.kernel(
        _embed_body,
        mesh=mesh,
        compiler_params=pltpu.CompilerParams(
            use_tc_tiling_on_sc=True, needs_layout_passes=False),
        out_type=jax.ShapeDtypeStruct((_BATCH + 1, _PAD_DIM), jnp.float32),
        scratch_types=[
            pltpu.VMEM((_BATCH,), jnp.int32),
            pltpu.VMEM((_CAP,), jnp.int32),
            pltpu.VMEM((_CAP,), jnp.int32),
            pltpu.VMEM((8, 8, _CHW), jnp.float32),
            pltpu.VMEM((8, 8, _CHW), jnp.float32),
            pltpu.VMEM((2, 16, _PAD_DIM), jnp.float32),
            pltpu.VMEM((2, 16), jnp.int32),
            pltpu.VMEM((_EMBED_DIM,), jnp.float32),
            pltpu.VMEM((_EMBED_DIM,), jnp.float32),
            pltpu.SemaphoreType.DMA,
            pltpu.SemaphoreType.DMA,
        ],
    )
    tab3 = table.T.reshape(8, 8, _NUM_MAPPERS + 1)
    out2 = f(mapper_ids, tab3, ln_gamma, ln_beta)
    return out2[:_BATCH, :_EMBED_DIM]


def kernel(mapper_ids, table, ln_gamma, ln_beta):
    return _embed(mapper_ids, table, ln_gamma, ln_beta)
